# baseline (speedup 1.0000x reference)
import functools

import jax
import jax.numpy as jnp
from jax import lax
from jax.experimental import pallas as pl
from jax.experimental.pallas import tpu as pltpu
from jax.experimental.pallas import tpu_sc as plsc

B = 16384
F = 32
M = 128
NC = 2
NS = 16
NW = NC * NS
BPW = B // NW
CH = 128
NCH = BPW // CH


def _sc_gather_body(user_hbm, item_hbm, tug, tig, tum, tim,
                    oug, oig, oum, oim,
                    idx_u, idx_i, gu, gi, mb,
                    sgu, sgi, smb):
    wid = lax.axis_index("s") * NC + lax.axis_index("c")
    base = wid * BPW
    row0 = wid * NCH
    pltpu.sync_copy(user_hbm.at[pl.ds(row0, NCH)], idx_u)
    pltpu.sync_copy(item_hbm.at[pl.ds(row0, NCH)], idx_i)

    cu = [pltpu.async_copy(tug.at[idx_u.at[j]], gu.at[j], sgu.at[j])
          for j in range(NCH)]
    ci = [pltpu.async_copy(tig.at[idx_i.at[j]], gi.at[j], sgi.at[j])
          for j in range(NCH)]
    cm = [pltpu.async_copy(tum.at[idx_u.at[j]], mb.at[j], smb.at[j])
          for j in range(NCH)]
    for j in range(NCH):
        cu[j].wait()
        pltpu.sync_copy(gu.at[j], oug.at[pl.ds(base + j * CH, CH)])
    for j in range(NCH):
        ci[j].wait()
        pltpu.sync_copy(gi.at[j], oig.at[pl.ds(base + j * CH, CH)])
    for j in range(NCH):
        cm[j].wait()
        pltpu.sync_copy(mb.at[j], oum.at[pl.ds(base + j * CH, CH)])
    cm2 = [pltpu.async_copy(tim.at[idx_i.at[j]], mb.at[j], smb.at[j])
           for j in range(NCH)]
    for j in range(NCH):
        cm2[j].wait()
        pltpu.sync_copy(mb.at[j], oim.at[pl.ds(base + j * CH, CH)])


@jax.jit
def _sc_gather(user2d, item2d, tug, tig, tum, tim):
    f32 = jnp.float32
    return pl.kernel(
        _sc_gather_body,
        out_type=(
            jax.ShapeDtypeStruct((B, F), f32),
            jax.ShapeDtypeStruct((B, F), f32),
            jax.ShapeDtypeStruct((B, M), f32),
            jax.ShapeDtypeStruct((B, M), f32),
        ),
        mesh=plsc.VectorSubcoreMesh(
            core_axis_name="c", subcore_axis_name="s",
            num_cores=NC, num_subcores=NS),
        compiler_params=pltpu.CompilerParams(use_tc_tiling_on_sc=False),
        scratch_types=(
            pltpu.VMEM((NCH, CH), jnp.int32),
            pltpu.VMEM((NCH, CH), jnp.int32),
            pltpu.VMEM((NCH, CH, F), f32),
            pltpu.VMEM((NCH, CH, F), f32),
            pltpu.VMEM((NCH, CH, M), f32),
            pltpu.SemaphoreType.DMA((NCH,)),
            pltpu.SemaphoreType.DMA((NCH,)),
            pltpu.SemaphoreType.DMA((NCH,)),
        ),
    )(user2d, item2d, tug, tig, tum, tim)


def _tc_body(ug, ig, um, im, w1u, w1i, b1, w2, b2, w3, b3, wpg, wph, bp, out):
    hi = jax.lax.Precision.HIGHEST
    h = (jnp.dot(um[...], w1u[...], precision=hi)
         + jnp.dot(im[...], w1i[...], precision=hi) + b1[...])
    h = jnp.maximum(h, 0.0)
    h = jnp.maximum(jnp.dot(h, w2[...], precision=hi) + b2[...], 0.0)
    h = jnp.maximum(jnp.dot(h, w3[...], precision=hi) + b3[...], 0.0)
    g = ug[...] * ig[...]
    pred = (jnp.sum(g * wpg[...], axis=1)
            + jnp.sum(h * wph[...], axis=1) + bp[0, 0])
    out[...] = pred


@functools.partial(jax.jit, static_argnames=("blk",))
def _tc_mlp(ug, ig, um, im, w1u, w1i, b1, w2, b2, w3, b3, wpg, wph, bp,
            blk=2048):
    grid = (B // blk,)
    full = lambda shape: pl.BlockSpec(shape, lambda i: (0, 0))
    return pl.pallas_call(
        _tc_body,
        grid=grid,
        in_specs=[
            pl.BlockSpec((blk, F), lambda i: (i, 0)),
            pl.BlockSpec((blk, F), lambda i: (i, 0)),
            pl.BlockSpec((blk, M), lambda i: (i, 0)),
            pl.BlockSpec((blk, M), lambda i: (i, 0)),
            full((M, M)), full((M, M)), full((1, M)),
            full((M, M // 2)), full((1, M // 2)),
            full((M // 2, F)), full((1, F)),
            full((1, F)), full((1, F)), full((1, 1)),
        ],
        out_specs=pl.BlockSpec((blk,), lambda i: (i,)),
        out_shape=jax.ShapeDtypeStruct((B,), jnp.float32),
    )(ug, ig, um, im, w1u, w1i, b1, w2, b2, w3, b3, wpg, wph, bp)


def kernel(user, item, embed_user_GMF, embed_item_GMF, embed_user_MLP,
           embed_item_MLP, W1, b1, W2, b2, W3, b3, Wp, bp):
    user2d = user.astype(jnp.int32).reshape(NW * NCH, CH)
    item2d = item.astype(jnp.int32).reshape(NW * NCH, CH)
    ug, ig, um, im = _sc_gather(user2d, item2d, embed_user_GMF,
                                embed_item_GMF, embed_user_MLP,
                                embed_item_MLP)
    pred = _tc_mlp(ug, ig, um, im,
                   W1[:M], W1[M:], b1.reshape(1, M),
                   W2, b2.reshape(1, M // 2),
                   W3, b3.reshape(1, F),
                   Wp[:F, 0].reshape(1, F), Wp[F:, 0].reshape(1, F),
                   bp.reshape(1, 1))
    return pred


# speedup vs baseline: 1.0097x; 1.0097x over previous
import functools

import jax
import jax.numpy as jnp
from jax import lax
from jax.experimental import pallas as pl
from jax.experimental.pallas import tpu as pltpu
from jax.experimental.pallas import tpu_sc as plsc

B = 16384
F = 32
M = 128
NC = 2
NS = 16
NW = NC * NS
BPW = B // NW
CH = 128
NCH = BPW // CH

_MESH = dict(core_axis_name="c", subcore_axis_name="s",
             num_cores=NC, num_subcores=NS)


def _worker_base():
    wid = lax.axis_index("s") * NC + lax.axis_index("c")
    return wid, wid * BPW


def _sc_gmf_body(user3d, item3d, tug, tig, oug, oig,
                 idx_u, idx_i, gu, gi, sgu, sgi):
    wid, base = _worker_base()
    pltpu.sync_copy(user3d.at[wid], idx_u)
    pltpu.sync_copy(item3d.at[wid], idx_i)
    cu = [pltpu.async_copy(tug.at[idx_u.at[j]], gu.at[j], sgu.at[j])
          for j in range(NCH)]
    ci = [pltpu.async_copy(tig.at[idx_i.at[j]], gi.at[j], sgi.at[j])
          for j in range(NCH)]
    for j in range(NCH):
        cu[j].wait()
        pltpu.sync_copy(gu.at[j], oug.at[pl.ds(base + j * CH, CH)])
    for j in range(NCH):
        ci[j].wait()
        pltpu.sync_copy(gi.at[j], oig.at[pl.ds(base + j * CH, CH)])


def _sc_mlp_body(user3d, item3d, tum, tim, oum, oim,
                 idx_u, idx_i, mb, smb):
    wid, base = _worker_base()
    pltpu.sync_copy(user3d.at[wid], idx_u)
    pltpu.sync_copy(item3d.at[wid], idx_i)
    cu = [pltpu.async_copy(tum.at[idx_u.at[j]], mb.at[j], smb.at[j])
          for j in range(NCH)]
    for j in range(NCH):
        cu[j].wait()
        pltpu.sync_copy(mb.at[j], oum.at[pl.ds(base + j * CH, CH)])
    ci = [pltpu.async_copy(tim.at[idx_i.at[j]], mb.at[j], smb.at[j])
          for j in range(NCH)]
    for j in range(NCH):
        ci[j].wait()
        pltpu.sync_copy(mb.at[j], oim.at[pl.ds(base + j * CH, CH)])


@jax.jit
def _sc_gather(user3d, item3d, tug, tig, tum, tim):
    f32 = jnp.float32
    ug, ig = pl.kernel(
        _sc_gmf_body,
        out_type=(
            jax.ShapeDtypeStruct((B, F), f32),
            jax.ShapeDtypeStruct((B, F), f32),
        ),
        mesh=plsc.VectorSubcoreMesh(**_MESH),
        compiler_params=pltpu.CompilerParams(use_tc_tiling_on_sc=False),
        scratch_types=(
            pltpu.VMEM((NCH, CH), jnp.int32),
            pltpu.VMEM((NCH, CH), jnp.int32),
            pltpu.VMEM((NCH, CH, F), f32),
            pltpu.VMEM((NCH, CH, F), f32),
            pltpu.SemaphoreType.DMA((NCH,)),
            pltpu.SemaphoreType.DMA((NCH,)),
        ),
    )(user3d, item3d, tug, tig)
    um, im = pl.kernel(
        _sc_mlp_body,
        out_type=(
            jax.ShapeDtypeStruct((B, M), f32),
            jax.ShapeDtypeStruct((B, M), f32),
        ),
        mesh=plsc.VectorSubcoreMesh(**_MESH),
        scratch_types=(
            pltpu.VMEM((NCH, CH), jnp.int32),
            pltpu.VMEM((NCH, CH), jnp.int32),
            pltpu.VMEM((NCH, CH, M), f32),
            pltpu.SemaphoreType.DMA((NCH,)),
        ),
    )(user3d, item3d, tum, tim)
    return ug, ig, um, im


def _tc_body(ug, ig, um, im, w1u, w1i, b1, w2, b2, w3, b3, wpg, wph, bp, out):
    hi = jax.lax.Precision.HIGHEST
    h = (jnp.dot(um[...], w1u[...], precision=hi)
         + jnp.dot(im[...], w1i[...], precision=hi) + b1[...])
    h = jnp.maximum(h, 0.0)
    h = jnp.maximum(jnp.dot(h, w2[...], precision=hi) + b2[...], 0.0)
    h = jnp.maximum(jnp.dot(h, w3[...], precision=hi) + b3[...], 0.0)
    g = ug[...] * ig[...]
    pred = (jnp.sum(g * wpg[...], axis=1)
            + jnp.sum(h * wph[...], axis=1) + bp[0, 0])
    out[...] = pred


@functools.partial(jax.jit, static_argnames=("blk",))
def _tc_mlp(ug, ig, um, im, w1u, w1i, b1, w2, b2, w3, b3, wpg, wph, bp,
            blk=2048):
    grid = (B // blk,)
    full = lambda shape: pl.BlockSpec(shape, lambda i: (0, 0))
    return pl.pallas_call(
        _tc_body,
        grid=grid,
        in_specs=[
            pl.BlockSpec((blk, F), lambda i: (i, 0)),
            pl.BlockSpec((blk, F), lambda i: (i, 0)),
            pl.BlockSpec((blk, M), lambda i: (i, 0)),
            pl.BlockSpec((blk, M), lambda i: (i, 0)),
            full((M, M)), full((M, M)), full((1, M)),
            full((M, M // 2)), full((1, M // 2)),
            full((M // 2, F)), full((1, F)),
            full((1, F)), full((1, F)), full((1, 1)),
        ],
        out_specs=pl.BlockSpec((blk,), lambda i: (i,)),
        out_shape=jax.ShapeDtypeStruct((B,), jnp.float32),
    )(ug, ig, um, im, w1u, w1i, b1, w2, b2, w3, b3, wpg, wph, bp)


def kernel(user, item, embed_user_GMF, embed_item_GMF, embed_user_MLP,
           embed_item_MLP, W1, b1, W2, b2, W3, b3, Wp, bp):
    user3d = user.astype(jnp.int32).reshape(NW, NCH, CH)
    item3d = item.astype(jnp.int32).reshape(NW, NCH, CH)
    ug, ig, um, im = _sc_gather(user3d, item3d, embed_user_GMF,
                                embed_item_GMF, embed_user_MLP,
                                embed_item_MLP)
    pred = _tc_mlp(ug, ig, um, im,
                   W1[:M], W1[M:], b1.reshape(1, M),
                   W2, b2.reshape(1, M // 2),
                   W3, b3.reshape(1, F),
                   Wp[:F, 0].reshape(1, F), Wp[F:, 0].reshape(1, F),
                   bp.reshape(1, 1))
    return pred


# speedup vs baseline: 1.2704x; 1.2583x over previous
import functools

import jax
import jax.numpy as jnp
from jax import lax
from jax.experimental import pallas as pl
from jax.experimental.pallas import tpu as pltpu
from jax.experimental.pallas import tpu_sc as plsc

B = 16384
F = 32
M = 128
NC = 2
NS = 16
NW = NC * NS
BPW = B // NW
CH = 128
NCH = BPW // CH

_MESH = dict(core_axis_name="c", subcore_axis_name="s",
             num_cores=NC, num_subcores=NS)


def _worker_base():
    wid = lax.axis_index("s") * NC + lax.axis_index("c")
    return wid, wid * BPW


def _sc_gmf_body(user3d, item3d, tug, tig, oug, oig,
                 idx_u, idx_i, gu, gi, sgu, sgi):
    wid, base = _worker_base()
    pltpu.sync_copy(user3d.at[wid], idx_u)
    pltpu.sync_copy(item3d.at[wid], idx_i)
    cu = [pltpu.async_copy(tug.at[idx_u.at[j]], gu.at[j], sgu.at[j])
          for j in range(NCH)]
    ci = [pltpu.async_copy(tig.at[idx_i.at[j]], gi.at[j], sgi.at[j])
          for j in range(NCH)]
    for j in range(NCH):
        cu[j].wait()
        pltpu.sync_copy(gu.at[j], oug.at[pl.ds(base + j * CH, CH)])
    for j in range(NCH):
        ci[j].wait()
        pltpu.sync_copy(gi.at[j], oig.at[pl.ds(base + j * CH, CH)])


def _sc_mlp_body(user3d, item3d, tum, tim, oum, oim,
                 idx_u, idx_i, mb, smb):
    wid, base = _worker_base()
    pltpu.sync_copy(user3d.at[wid], idx_u)
    pltpu.sync_copy(item3d.at[wid], idx_i)
    cu = [pltpu.async_copy(tum.at[idx_u.at[j]], mb.at[j], smb.at[j])
          for j in range(NCH)]
    for j in range(NCH):
        cu[j].wait()
        pltpu.sync_copy(mb.at[j], oum.at[pl.ds(base + j * CH, CH)])
    ci = [pltpu.async_copy(tim.at[idx_i.at[j]], mb.at[j], smb.at[j])
          for j in range(NCH)]
    for j in range(NCH):
        ci[j].wait()
        pltpu.sync_copy(mb.at[j], oim.at[pl.ds(base + j * CH, CH)])


@jax.jit
def _sc_gather(user3d, item3d, tug, tig, tum, tim):
    f32 = jnp.float32
    um, im = pl.kernel(
        _sc_mlp_body,
        out_type=(
            jax.ShapeDtypeStruct((B, M), f32),
            jax.ShapeDtypeStruct((B, M), f32),
        ),
        mesh=plsc.VectorSubcoreMesh(**_MESH),
        scratch_types=(
            pltpu.VMEM((NCH, CH), jnp.int32),
            pltpu.VMEM((NCH, CH), jnp.int32),
            pltpu.VMEM((NCH, CH, M), f32),
            pltpu.SemaphoreType.DMA((NCH,)),
        ),
    )(user3d, item3d, tum, tim)
    ug, ig = pl.kernel(
        _sc_gmf_body,
        out_type=(
            jax.ShapeDtypeStruct((B, F), f32),
            jax.ShapeDtypeStruct((B, F), f32),
        ),
        mesh=plsc.VectorSubcoreMesh(**_MESH),
        compiler_params=pltpu.CompilerParams(use_tc_tiling_on_sc=False),
        scratch_types=(
            pltpu.VMEM((NCH, CH), jnp.int32),
            pltpu.VMEM((NCH, CH), jnp.int32),
            pltpu.VMEM((NCH, CH, F), f32),
            pltpu.VMEM((NCH, CH, F), f32),
            pltpu.SemaphoreType.DMA((NCH,)),
            pltpu.SemaphoreType.DMA((NCH,)),
        ),
    )(user3d, item3d, tug, tig)
    return ug, ig, um, im


def _tc_body(ug, ig, um, im, w1u, w1i, b1, w2, b2, w3, b3, wpg, wph, bp, out):
    hi = jax.lax.Precision.DEFAULT
    h = (jnp.dot(um[...], w1u[...], precision=hi)
         + jnp.dot(im[...], w1i[...], precision=hi) + b1[...])
    h = jnp.maximum(h, 0.0)
    h = jnp.maximum(jnp.dot(h, w2[...], precision=hi) + b2[...], 0.0)
    h = jnp.maximum(jnp.dot(h, w3[...], precision=hi) + b3[...], 0.0)
    g = ug[...] * ig[...]
    pred = (jnp.sum(g * wpg[...], axis=1)
            + jnp.sum(h * wph[...], axis=1) + bp[0, 0])
    out[...] = pred


@functools.partial(jax.jit, static_argnames=("blk",))
def _tc_mlp(ug, ig, um, im, w1u, w1i, b1, w2, b2, w3, b3, wpg, wph, bp,
            blk=2048):
    grid = (B // blk,)
    full = lambda shape: pl.BlockSpec(shape, lambda i: (0, 0))
    return pl.pallas_call(
        _tc_body,
        grid=grid,
        in_specs=[
            pl.BlockSpec((blk, F), lambda i: (i, 0)),
            pl.BlockSpec((blk, F), lambda i: (i, 0)),
            pl.BlockSpec((blk, M), lambda i: (i, 0)),
            pl.BlockSpec((blk, M), lambda i: (i, 0)),
            full((M, M)), full((M, M)), full((1, M)),
            full((M, M // 2)), full((1, M // 2)),
            full((M // 2, F)), full((1, F)),
            full((1, F)), full((1, F)), full((1, 1)),
        ],
        out_specs=pl.BlockSpec((blk,), lambda i: (i,)),
        out_shape=jax.ShapeDtypeStruct((B,), jnp.float32),
    )(ug, ig, um, im, w1u, w1i, b1, w2, b2, w3, b3, wpg, wph, bp)


def kernel(user, item, embed_user_GMF, embed_item_GMF, embed_user_MLP,
           embed_item_MLP, W1, b1, W2, b2, W3, b3, Wp, bp):
    user3d = user.astype(jnp.int32).reshape(NW, NCH, CH)
    item3d = item.astype(jnp.int32).reshape(NW, NCH, CH)
    ug, ig, um, im = _sc_gather(user3d, item3d, embed_user_GMF,
                                embed_item_GMF, embed_user_MLP,
                                embed_item_MLP)
    pred = _tc_mlp(ug, ig, um, im,
                   W1[:M], W1[M:], b1.reshape(1, M),
                   W2, b2.reshape(1, M // 2),
                   W3, b3.reshape(1, F),
                   Wp[:F, 0].reshape(1, F), Wp[F:, 0].reshape(1, F),
                   bp.reshape(1, 1))
    return pred
